# Initial kernel scaffold; baseline (speedup 1.0000x reference)
import jax
import jax.numpy as jnp
from jax.experimental import pallas as pl


def kernel(A, h, W_conv, gcn_w, gcn_b, lin1_w, lin1_b, lin2_w, lin2_b):
    raise NotImplementedError("write your pallas kernel here")


# speedup vs baseline: 4.2491x; 4.2491x over previous
import functools

import jax
import jax.numpy as jnp
from jax.experimental import pallas as pl
from jax.experimental.pallas import tpu as pltpu

N = 2048
NUM_EDGE = 5
NUM_CHANNELS = 2
IN_DIM = 256
HIDDEN = 64
NUM_CLASS = 8
WIDE = 384
BM = 256


def _combine(filt_ref, a_ref, c):
    acc = filt_ref[c, 0] * a_ref[0]
    for r in range(1, NUM_EDGE):
        acc = acc + filt_ref[c, r] * a_ref[r]
    return acc


def _stage_kernel(filt_ref, a_ref, t_ref, out_ref):
    for c in range(NUM_CHANNELS):
        ac = _combine(filt_ref, a_ref, c)
        out_ref[c] = jnp.dot(ac, t_ref[c], preferred_element_type=jnp.float32)


def _final_kernel(filt_ref, a_ref, t_ref, gw_ref, gb_ref, l1w_ref, l1b_ref,
                  l2w_ref, l2b_ref, y_ref):
    xs = []
    for c in range(NUM_CHANNELS):
        ac = _combine(filt_ref, a_ref, c)
        v = jnp.dot(ac, t_ref[c], preferred_element_type=jnp.float32)
        num = v[:, :IN_DIM]
        e = v[:, IN_DIM:IN_DIM + 1]
        agg = num / jnp.where(e == 0.0, 1.0, e)
        x = jnp.dot(agg, gw_ref[...], preferred_element_type=jnp.float32)
        xs.append(jnp.maximum(x + gb_ref[...], 0.0))
    z = (jnp.dot(xs[0], l1w_ref[:HIDDEN], preferred_element_type=jnp.float32)
         + jnp.dot(xs[1], l1w_ref[HIDDEN:], preferred_element_type=jnp.float32)
         + l1b_ref[...])
    z = jnp.maximum(z, 0.0)
    y_ref[...] = (jnp.dot(z, l2w_ref[...], preferred_element_type=jnp.float32)
                  + l2b_ref[...])


def _stage(filt, A, t):
    return pl.pallas_call(
        _stage_kernel,
        grid=(N // BM,),
        in_specs=[
            pl.BlockSpec(memory_space=pltpu.SMEM),
            pl.BlockSpec((NUM_EDGE, BM, N), lambda i: (0, i, 0)),
            pl.BlockSpec((NUM_CHANNELS, N, WIDE), lambda i: (0, 0, 0)),
        ],
        out_specs=pl.BlockSpec((NUM_CHANNELS, BM, WIDE), lambda i: (0, i, 0)),
        out_shape=jax.ShapeDtypeStruct((NUM_CHANNELS, N, WIDE), jnp.float32),
    )(filt, A, t)


def _final(filt, A, t, gcn_w, gcn_b, lin1_w, lin1_b, lin2_w, lin2_b):
    small = lambda shp: pl.BlockSpec(shp, lambda i: tuple(0 for _ in shp))
    return pl.pallas_call(
        _final_kernel,
        grid=(N // BM,),
        in_specs=[
            pl.BlockSpec(memory_space=pltpu.SMEM),
            pl.BlockSpec((NUM_EDGE, BM, N), lambda i: (0, i, 0)),
            pl.BlockSpec((NUM_CHANNELS, N, WIDE), lambda i: (0, 0, 0)),
            small((IN_DIM, HIDDEN)),
            small((1, HIDDEN)),
            small((NUM_CHANNELS * HIDDEN, HIDDEN)),
            small((1, HIDDEN)),
            small((HIDDEN, NUM_CLASS)),
            small((1, NUM_CLASS)),
        ],
        out_specs=pl.BlockSpec((BM, NUM_CLASS), lambda i: (i, 0)),
        out_shape=jax.ShapeDtypeStruct((N, NUM_CLASS), jnp.float32),
    )(filt, A, t, gcn_w, gcn_b, lin1_w, lin1_b, lin2_w, lin2_b)


def kernel(A, h, W_conv, gcn_w, gcn_b, lin1_w, lin1_b, lin2_w, lin2_b):
    filt = jax.nn.softmax(W_conv, axis=2)
    t0 = jnp.concatenate(
        [h, jnp.ones((N, 1), jnp.float32), jnp.zeros((N, WIDE - IN_DIM - 1), jnp.float32)],
        axis=1)
    t = jnp.stack([t0] * NUM_CHANNELS)
    t = _stage(filt[2], A, t)
    t = _stage(filt[1], A, t)
    return _final(filt[0], A, t,
                  gcn_w, gcn_b.reshape(1, HIDDEN),
                  lin1_w, lin1_b.reshape(1, HIDDEN),
                  lin2_w, lin2_b.reshape(1, NUM_CLASS))
